# baseline (speedup 1.0000x reference)
import functools

import jax
import jax.numpy as jnp
from jax import lax
from jax.experimental import pallas as pl
from jax.experimental.pallas import tpu as pltpu
from jax.experimental.pallas import tpu_sc as plsc

VOCAB_NUM = 100000
SEQ = 4096
BATCH = 16
TOTAL = 32768
HALF = SEQ // 2
LANES = 16
CHUNKS = HALF // LANES
WIN = HALF + 8
BASE_MAX = TOTAL - WIN
REM_MAX = TOTAL + HALF - BASE_MAX
BUF = ((REM_MAX + HALF + 15) // 16) * 16


def _body(flat_hbm, cu_hbm, out_hbm, cu_v, buf_v, row_v):
    c = lax.axis_index("c")
    s = lax.axis_index("s")
    wid = c * 16 + s
    b = wid // 2
    h = wid % 2

    pltpu.sync_copy(cu_hbm, cu_v.at[pl.ds(0, BATCH + 1)])

    v = cu_v[pl.ds(b, LANES)]
    start = v[0]
    end = v[1]
    seg_len = jnp.minimum(end - start, SEQ)

    src0 = start + h * HALF
    base = jnp.minimum(jnp.bitwise_and(src0, jnp.int32(-8)),
                       jnp.int32(BASE_MAX))
    base = pl.multiple_of(base, 8)
    rem = src0 - base

    pltpu.sync_copy(flat_hbm.at[pl.ds(base, WIN)], buf_v.at[pl.ds(0, WIN)])

    col0 = h * HALF
    lanes = lax.iota(jnp.int32, LANES)

    UNROLL = 8
    inv_v = jnp.float32(1.0 / VOCAB_NUM)
    voc = jnp.int32(VOCAB_NUM)

    def step(k0, _):
        for j in range(UNROLL):
            k = k0 * UNROLL + j
            chunk = buf_v[pl.ds(rem + k * LANES, LANES)]
            q = (chunk.astype(jnp.float32) * inv_v).astype(jnp.int32)
            r = chunk - q * voc
            r = jnp.where(r < 0, r + voc, r)
            hashed = jnp.where(r >= voc, r - voc, r)
            col = col0 + k * LANES + lanes
            row_v[pl.ds(k * LANES, LANES)] = jnp.where(col < seg_len, hashed, 0)
        return _

    lax.fori_loop(0, CHUNKS // UNROLL, step, None)

    pltpu.sync_copy(row_v, out_hbm.at[pl.ds(b * SEQ + col0, HALF)])


@functools.cache
def _get_packer():
    mesh = plsc.VectorSubcoreMesh(core_axis_name="c", subcore_axis_name="s")
    return functools.partial(
        pl.kernel,
        out_type=jax.ShapeDtypeStruct((BATCH * SEQ,), jnp.int32),
        mesh=mesh,
        compiler_params=pltpu.CompilerParams(
            disable_bounds_checks=True,
            disable_semaphore_checks=True,
            skip_device_barrier=True,
        ),
        scratch_types=[
            pltpu.VMEM((2 * LANES,), jnp.int32),
            pltpu.VMEM((BUF,), jnp.int32),
            pltpu.VMEM((HALF,), jnp.int32),
        ],
    )(_body)


def kernel(flat, cu_seqlens):
    out_dtype = flat.dtype
    out = _get_packer()(flat.astype(jnp.int32),
                        cu_seqlens.astype(jnp.int32))
    return out.reshape(BATCH, SEQ).astype(out_dtype)


# speedup vs baseline: 1.0134x; 1.0134x over previous
import functools

import jax
import jax.numpy as jnp
from jax import lax
from jax.experimental import pallas as pl
from jax.experimental.pallas import tpu as pltpu
from jax.experimental.pallas import tpu_sc as plsc

VOCAB_NUM = 100000
SEQ = 4096
BATCH = 16
TOTAL = 32768
HALF = SEQ // 2
LANES = 16
CHUNKS = HALF // LANES
WIN = HALF + 8
BASE_MAX = TOTAL - WIN
REM_MAX = TOTAL + HALF - BASE_MAX
BUF = ((REM_MAX + HALF + 15) // 16) * 16


def _body(flat_hbm, cu_hbm, out_hbm, cu_v, buf_v, row_v):
    c = lax.axis_index("c")
    s = lax.axis_index("s")
    wid = c * 16 + s
    b = wid // 2
    h = wid % 2

    pltpu.sync_copy(cu_hbm, cu_v.at[pl.ds(0, BATCH + 1)])

    v = cu_v[pl.ds(b, LANES)]
    start = v[0]
    end = v[1]
    seg_len = jnp.minimum(end - start, SEQ)

    src0 = start + h * HALF
    base = jnp.minimum(jnp.bitwise_and(src0, jnp.int32(-8)),
                       jnp.int32(BASE_MAX))
    base = pl.multiple_of(base, 8)
    rem = src0 - base

    pltpu.sync_copy(flat_hbm.at[pl.ds(base, WIN)], buf_v.at[pl.ds(0, WIN)])

    col0 = h * HALF
    lanes = lax.iota(jnp.int32, LANES)

    UNROLL = 4
    inv_v = jnp.float32(1.0 / VOCAB_NUM)
    voc = jnp.int32(VOCAB_NUM)

    def step(k0, _):
        for j in range(UNROLL):
            k = k0 * UNROLL + j
            chunk = buf_v[pl.ds(rem + k * LANES, LANES)]
            q = (chunk.astype(jnp.float32) * inv_v).astype(jnp.int32)
            r = chunk - q * voc
            r = jnp.where(r < 0, r + voc, r)
            hashed = jnp.where(r >= voc, r - voc, r)
            col = col0 + k * LANES + lanes
            row_v[pl.ds(k * LANES, LANES)] = jnp.where(col < seg_len, hashed, 0)
        return _

    lax.fori_loop(0, CHUNKS // UNROLL, step, None)

    pltpu.sync_copy(row_v, out_hbm.at[pl.ds(b * SEQ + col0, HALF)])


@functools.cache
def _get_packer():
    mesh = plsc.VectorSubcoreMesh(core_axis_name="c", subcore_axis_name="s")
    return functools.partial(
        pl.kernel,
        out_type=jax.ShapeDtypeStruct((BATCH * SEQ,), jnp.int32),
        mesh=mesh,
        scratch_types=[
            pltpu.VMEM((2 * LANES,), jnp.int32),
            pltpu.VMEM((BUF,), jnp.int32),
            pltpu.VMEM((HALF,), jnp.int32),
        ],
    )(_body)


def kernel(flat, cu_seqlens):
    out_dtype = flat.dtype
    out = _get_packer()(flat.astype(jnp.int32),
                        cu_seqlens.astype(jnp.int32))
    return out.reshape(BATCH, SEQ).astype(out_dtype)


# speedup vs baseline: 1.0160x; 1.0026x over previous
import functools

import jax
import jax.numpy as jnp
from jax import lax
from jax.experimental import pallas as pl
from jax.experimental.pallas import tpu as pltpu
from jax.experimental.pallas import tpu_sc as plsc

VOCAB_NUM = 100000
SEQ = 4096
BATCH = 16
TOTAL = 32768
HALF = SEQ // 2
LANES = 16
CHUNKS = HALF // LANES
WIN = HALF + 8
BASE_MAX = TOTAL - WIN
REM_MAX = TOTAL + HALF - BASE_MAX
BUF = ((REM_MAX + HALF + 15) // 16) * 16


def _body(flat_hbm, cu_hbm, out_hbm, cu_v, buf_v, row_v):
    c = lax.axis_index("c")
    s = lax.axis_index("s")
    wid = c * 16 + s
    b = wid // 2
    h = wid % 2

    pltpu.sync_copy(cu_hbm, cu_v.at[pl.ds(0, BATCH + 1)])

    v = cu_v[pl.ds(b, LANES)]
    start = v[0]
    end = v[1]
    seg_len = jnp.minimum(end - start, SEQ)

    src0 = start + h * HALF
    base = jnp.minimum(jnp.bitwise_and(src0, jnp.int32(-8)),
                       jnp.int32(BASE_MAX))
    base = pl.multiple_of(base, 8)
    rem = src0 - base

    pltpu.sync_copy(flat_hbm.at[pl.ds(base, WIN)], buf_v.at[pl.ds(0, WIN)])

    col0 = h * HALF
    lanes = lax.iota(jnp.int32, LANES)

    UNROLL = 2
    inv_v = jnp.float32(1.0 / VOCAB_NUM)
    voc = jnp.int32(VOCAB_NUM)

    def step(k0, _):
        for j in range(UNROLL):
            k = k0 * UNROLL + j
            chunk = buf_v[pl.ds(rem + k * LANES, LANES)]
            q = (chunk.astype(jnp.float32) * inv_v).astype(jnp.int32)
            r = chunk - q * voc
            r = jnp.where(r < 0, r + voc, r)
            hashed = jnp.where(r >= voc, r - voc, r)
            col = col0 + k * LANES + lanes
            row_v[pl.ds(k * LANES, LANES)] = jnp.where(col < seg_len, hashed, 0)
        return _

    lax.fori_loop(0, CHUNKS // UNROLL, step, None)

    pltpu.sync_copy(row_v, out_hbm.at[pl.ds(b * SEQ + col0, HALF)])


@functools.cache
def _get_packer():
    mesh = plsc.VectorSubcoreMesh(core_axis_name="c", subcore_axis_name="s")
    return functools.partial(
        pl.kernel,
        out_type=jax.ShapeDtypeStruct((BATCH * SEQ,), jnp.int32),
        mesh=mesh,
        scratch_types=[
            pltpu.VMEM((2 * LANES,), jnp.int32),
            pltpu.VMEM((BUF,), jnp.int32),
            pltpu.VMEM((HALF,), jnp.int32),
        ],
    )(_body)


def kernel(flat, cu_seqlens):
    out_dtype = flat.dtype
    out = _get_packer()(flat.astype(jnp.int32),
                        cu_seqlens.astype(jnp.int32))
    return out.reshape(BATCH, SEQ).astype(out_dtype)
